# baseline (speedup 1.0000x reference)
import functools

import jax
import jax.numpy as jnp
from jax import lax
from jax.experimental import pallas as pl
from jax.experimental.pallas import tpu as pltpu
from jax.experimental.pallas import tpu_sc as plsc

H = 128
L = 16
NCORES = 2
NSUB = 16
NW = NCORES * NSUB
CHUNK = 40
GC = 10


def _mm_body_f32(x_ref, w_ref, b_ref, o_ref):
    o_ref[...] = (
        jnp.dot(x_ref[...], w_ref[...], preferred_element_type=jnp.float32)
        + b_ref[...]
    )


def _mm_body_packed(x_ref, w_ref, b_ref, o_ref):
    y = (jnp.dot(x_ref[...], w_ref[...], preferred_element_type=jnp.float32)
         + b_ref[...])
    lo = lax.bitcast_convert_type(y[:, :H // 2].astype(jnp.bfloat16),
                                  jnp.uint16)
    hi = lax.bitcast_convert_type(y[:, H // 2:].astype(jnp.bfloat16),
                                  jnp.uint16)
    word = lo.astype(jnp.uint32) | (hi.astype(jnp.uint32) << 16)
    o_ref[...] = lax.bitcast_convert_type(word, jnp.int32)


def _mm_bias(x, w_full, w_half, b2d, block_rows, packed):
    m = x.shape[0]
    assert m % block_rows == 0
    body = _mm_body_packed if packed else _mm_body_f32
    ow = H // 2 if packed else H
    odt = jnp.int32 if packed else jnp.float32
    return pl.pallas_call(
        body,
        grid=(m // block_rows,),
        in_specs=[
            pl.BlockSpec((block_rows, H), lambda i: (i, 0)),
            pl.BlockSpec((H, H), lambda i, h=w_half: (h, 0)),
            pl.BlockSpec((1, H), lambda i: (0, 0)),
        ],
        out_specs=pl.BlockSpec((block_rows, ow), lambda i: (i, 0)),
        out_shape=jax.ShapeDtypeStruct((m, ow), odt),
    )(x, w_full, b2d)


def _sc_edge_kernel(n_nodes, n_edges):
    per_w = n_edges // NW
    n_groups = per_w // (CHUNK * GC)
    assert per_w == n_groups * CHUNK * GC, "edge partition must be exact"
    rows_per_tile = -(-n_nodes // NSUB)
    rows_per_tile = -(-rows_per_tile // 8) * 8
    n_pad = rows_per_tile * NSUB

    mesh = plsc.VectorSubcoreMesh(core_axis_name="c", subcore_axis_name="s")

    @functools.partial(
        pl.kernel,
        mesh=mesh,
        out_type=[
            jax.ShapeDtypeStruct((NCORES, n_pad, H), jnp.float32),
            jax.ShapeDtypeStruct((NCORES * n_pad,), jnp.float32),
        ],
        scratch_types=[
            pltpu.VMEM((CHUNK, H // 2), jnp.int32),
            pltpu.VMEM((CHUNK, H // 2), jnp.int32),
            pltpu.VMEM((CHUNK, H), jnp.float32),
            pltpu.VMEM((CHUNK, H), jnp.float32),
            pltpu.VMEM((CHUNK, H), jnp.float32),
            pltpu.VMEM((CHUNK, H), jnp.float32),
            pltpu.VMEM((CHUNK, H), jnp.float32),
            pltpu.VMEM((CHUNK, H), jnp.float32),
            pltpu.VMEM((GC, CHUNK), jnp.int32),
            pltpu.VMEM((GC, CHUNK), jnp.int32),
            pltpu.VMEM(((CHUNK // L + 1) * L,), jnp.float32),
            pltpu.VMEM((rows_per_tile + L,), jnp.float32),
            pltpu.VMEM_SHARED((n_pad, H), jnp.float32),
            pltpu.VMEM_SHARED((n_pad,), jnp.float32),
            pltpu.SemaphoreType.DMA,
            pltpu.SemaphoreType.DMA,
            pltpu.SemaphoreType.DMA,
            pltpu.SemaphoreType.DMA,
            pltpu.SemaphoreType.DMA,
            pltpu.SemaphoreType.DMA,
            pltpu.SemaphoreType.DMA,
            pltpu.SemaphoreType.DMA,
            pltpu.SemaphoreType.DMA,
            pltpu.SemaphoreType.DMA,
            pltpu.SemaphoreType.DMA,
            pltpu.SemaphoreType.DMA,
        ],
    )
    def sc_k(p_hbm, q_hbm, row_hbm, col_hbm, acc_out, cnt_out,
             q0, q1, a0, a1, a2, b0, b1, b2, ir2d, ic2d, ones, zc,
             acc_sh, cnt_sh, sem_i, sem_q0, sem_q1,
             sem_a0, sem_a1, sem_a2, sem_b0, sem_b1, sem_b2,
             sem_s0, sem_s1, sem_s2):
        cid = lax.axis_index("c")
        sid = lax.axis_index("s")
        wid = cid * NSUB + sid
        ebase = wid * per_w
        r0 = sid * rows_per_tile
        qsets = ((q0, sem_q0), (q1, sem_q1))
        absets = ((a0, b0, sem_a0, sem_b0, sem_s0),
                  (a1, b1, sem_a1, sem_b1, sem_s1),
                  (a2, b2, sem_a2, sem_b2, sem_s2))

        def _zfill(e, carry):
            for g in range(H // L):
                a0[e, pl.ds(g * L, L)] = jnp.zeros((L,), jnp.float32)
            return carry
        lax.fori_loop(0, CHUNK, _zfill, 0)

        def _ofill(e, carry):
            ones[pl.ds(e * L, L)] = jnp.ones((L,), jnp.float32)
            return carry
        lax.fori_loop(0, CHUNK // L + 1, _ofill, 0)

        def _cfill(e, carry):
            zc[pl.ds(e * L, L)] = jnp.zeros((L,), jnp.float32)
            return carry
        lax.fori_loop(0, (rows_per_tile + L) // L, _cfill, 0)

        n_z = rows_per_tile // CHUNK
        for z in range(n_z):
            pltpu.sync_copy(a0, acc_sh.at[pl.ds(r0 + z * CHUNK, CHUNK)])
        rz = rows_per_tile - n_z * CHUNK
        if rz:
            pltpu.sync_copy(a0.at[pl.ds(0, rz)],
                            acc_sh.at[pl.ds(r0 + n_z * CHUNK, rz)])
        pltpu.sync_copy(zc.at[pl.ds(0, rows_per_tile)],
                        cnt_sh.at[pl.ds(r0, rows_per_tile)])
        plsc.subcore_barrier()

        def _start_q(gbase, j):
            qb, sq = qsets[j % 2]
            return (pltpu.async_copy(
                q_hbm.at[pl.ds(gbase + j * CHUNK, CHUNK)], qb, sq),)

        def _start_ab(gbase, j):
            ab, bb, sa, sb, _ = absets[j % 3]
            ca = pltpu.async_copy(p_hbm.at[ir2d.at[j]], ab, sa)
            cb = pltpu.async_copy(p_hbm.at[ic2d.at[j]], bb, sb)
            return (ca, cb)

        def _group(g, carry):
            gbase = ebase + g * (CHUNK * GC)
            idx_cps = []
            for j in range(GC):
                idx_cps.append(pltpu.async_copy(
                    row_hbm.at[pl.ds(gbase + j * CHUNK, CHUNK)],
                    ir2d.at[j], sem_i))
                idx_cps.append(pltpu.async_copy(
                    col_hbm.at[pl.ds(gbase + j * CHUNK, CHUNK)],
                    ic2d.at[j], sem_i))
            for cp in idx_cps:
                cp.wait()

            pend_q = [(), ()]
            pend_ab = [(), (), ()]
            pend_sc = [(), (), ()]
            pend_q[0] = _start_q(gbase, 0)
            pend_ab[0] = _start_ab(gbase, 0)
            pend_ab[1] = _start_ab(gbase, 1)
            for j in range(GC):
                s = j % 3
                qb, _ = qsets[j % 2]
                ab, bb, _, _, sem_s = absets[s]
                for cp in pend_q[j % 2]:
                    cp.wait()
                pend_q[j % 2] = ()
                for cp in pend_ab[s]:
                    cp.wait()
                pend_ab[s] = ()
                if j + 1 < GC:
                    pend_q[(j + 1) % 2] = _start_q(gbase, j + 1)
                if j + 2 < GC:
                    nxt = (j + 2) % 3
                    for cp in pend_sc[nxt]:
                        cp.wait()
                    pend_sc[nxt] = ()
                    pend_ab[nxt] = _start_ab(gbase, j + 2)

                @plsc.parallel_loop(0, CHUNK, 1, unroll=2)
                def body(e, qb=qb, ab=ab, bb=bb):
                    msk = jnp.int32(-65536)
                    for k in range(H // 2 // L):
                        wq = qb[e, pl.ds(k * L, L)]
                        qlo = lax.bitcast_convert_type(wq << 16, jnp.float32)
                        qhi = lax.bitcast_convert_type(wq & msk, jnp.float32)
                        slo = pl.ds(k * L, L)
                        shi = pl.ds(H // 2 + k * L, L)
                        ab[e, slo] = jnp.maximum(ab[e, slo] + qlo, 0.0)
                        ab[e, shi] = jnp.maximum(ab[e, shi] + qhi, 0.0)
                        bb[e, slo] = jnp.maximum(bb[e, slo] + qlo, 0.0)
                        bb[e, shi] = jnp.maximum(bb[e, shi] + qhi, 0.0)

                pend_sc[s] = (
                    pltpu.async_copy(ab, acc_sh.at[ic2d.at[j]], sem_s,
                                     add=True),
                    pltpu.async_copy(bb, acc_sh.at[ir2d.at[j]], sem_s,
                                     add=True),
                    pltpu.async_copy(ones.at[pl.ds(0, CHUNK)],
                                     cnt_sh.at[ic2d.at[j]], sem_s, add=True),
                    pltpu.async_copy(ones.at[pl.ds(0, CHUNK)],
                                     cnt_sh.at[ir2d.at[j]], sem_s, add=True),
                )
            for grp in pend_sc:
                for cp in grp:
                    cp.wait()
            return carry
        lax.fori_loop(0, n_groups, _group, 0)

        plsc.subcore_barrier()
        pltpu.sync_copy(acc_sh.at[pl.ds(r0, rows_per_tile)],
                        acc_out.at[cid, pl.ds(r0, rows_per_tile)])
        pltpu.sync_copy(cnt_sh.at[pl.ds(r0, rows_per_tile)],
                        zc.at[pl.ds(0, rows_per_tile)])
        pltpu.sync_copy(zc.at[pl.ds(0, rows_per_tile)],
                        cnt_out.at[pl.ds(cid * n_pad + r0, rows_per_tile)])

    return sc_k


def _fin_body(acc_ref, cnt_ref, nodes_ref, sc_ref, bi_ref, o_ref):
    acc = acc_ref[0] + acc_ref[1]
    cnt = cnt_ref[...]
    u = acc / jnp.maximum(cnt, 1.0) + nodes_ref[...]
    mu = jnp.mean(u, axis=1, keepdims=True)
    var = jnp.mean((u - mu) ** 2, axis=1, keepdims=True)
    o_ref[...] = (u - mu) * lax.rsqrt(var + 1e-6) * sc_ref[...] + bi_ref[...]


def _finalize(acc2, cnt1, nodes, scale2d, bias2d, block_rows):
    n = nodes.shape[0]
    assert n % block_rows == 0
    return pl.pallas_call(
        _fin_body,
        grid=(n // block_rows,),
        in_specs=[
            pl.BlockSpec((NCORES, block_rows, H), lambda i: (0, i, 0)),
            pl.BlockSpec((block_rows, 1), lambda i: (i, 0)),
            pl.BlockSpec((block_rows, H), lambda i: (i, 0)),
            pl.BlockSpec((1, H), lambda i: (0, 0)),
            pl.BlockSpec((1, H), lambda i: (0, 0)),
        ],
        out_specs=pl.BlockSpec((block_rows, H), lambda i: (i, 0)),
        out_shape=jax.ShapeDtypeStruct((n, H), jnp.float32),
    )(acc2, cnt1, nodes, scale2d, bias2d)


def kernel(node_features, edge_index, edge_features, W_msg, b_msg, ln_scale, ln_bias):
    nodes = node_features[0]
    feat = edge_features[0]
    row = edge_index[0, 0].astype(jnp.int32)
    col = edge_index[0, 1].astype(jnp.int32)
    n_nodes = nodes.shape[0]
    n_edges = feat.shape[0]

    b2d = b_msg.reshape(1, H)
    zb = jnp.zeros((1, H), jnp.float32)

    p = _mm_bias(nodes, W_msg, 0, b2d, 1000, packed=False)
    q = _mm_bias(feat, W_msg, 1, zb, 2000, packed=True)

    acc2, cnt2 = _sc_edge_kernel(n_nodes, n_edges)(p, q, row, col)
    n_pad = acc2.shape[1]
    cnt1 = (cnt2[:n_pad] + cnt2[n_pad:])[:, None]

    out = _finalize(acc2, cnt1, nodes, ln_scale.reshape(1, H),
                    ln_bias.reshape(1, H), 2000)
    return out[None]


# speedup vs baseline: 1.0554x; 1.0554x over previous
import functools

import jax
import jax.numpy as jnp
from jax import lax
from jax.experimental import pallas as pl
from jax.experimental.pallas import tpu as pltpu
from jax.experimental.pallas import tpu_sc as plsc

H = 128
L = 16
NCORES = 2
NSUB = 16
NW = NCORES * NSUB
CHUNK = 40
GC = 25


def _mm_body_f32(x_ref, w_ref, b_ref, o_ref):
    o_ref[...] = (
        jnp.dot(x_ref[...], w_ref[...], preferred_element_type=jnp.float32)
        + b_ref[...]
    )


def _mm_body_packed(x_ref, w_ref, b_ref, o_ref):
    y = (jnp.dot(x_ref[...], w_ref[...], preferred_element_type=jnp.float32)
         + b_ref[...])
    lo = lax.bitcast_convert_type(y[:, :H // 2].astype(jnp.bfloat16),
                                  jnp.uint16)
    hi = lax.bitcast_convert_type(y[:, H // 2:].astype(jnp.bfloat16),
                                  jnp.uint16)
    word = lo.astype(jnp.uint32) | (hi.astype(jnp.uint32) << 16)
    o_ref[...] = lax.bitcast_convert_type(word, jnp.int32)


def _mm_bias(x, w_full, w_half, b2d, block_rows, packed):
    m = x.shape[0]
    assert m % block_rows == 0
    body = _mm_body_packed if packed else _mm_body_f32
    ow = H // 2 if packed else H
    odt = jnp.int32 if packed else jnp.float32
    return pl.pallas_call(
        body,
        grid=(m // block_rows,),
        in_specs=[
            pl.BlockSpec((block_rows, H), lambda i: (i, 0)),
            pl.BlockSpec((H, H), lambda i, h=w_half: (h, 0)),
            pl.BlockSpec((1, H), lambda i: (0, 0)),
        ],
        out_specs=pl.BlockSpec((block_rows, ow), lambda i: (i, 0)),
        out_shape=jax.ShapeDtypeStruct((m, ow), odt),
    )(x, w_full, b2d)


def _sc_edge_kernel(n_nodes, n_edges):
    per_w = n_edges // NW
    n_groups = per_w // (CHUNK * GC)
    assert per_w == n_groups * CHUNK * GC, "edge partition must be exact"
    rows_per_tile = -(-n_nodes // NSUB)
    rows_per_tile = -(-rows_per_tile // 8) * 8
    n_pad = rows_per_tile * NSUB

    mesh = plsc.VectorSubcoreMesh(core_axis_name="c", subcore_axis_name="s")

    @functools.partial(
        pl.kernel,
        mesh=mesh,
        out_type=[
            jax.ShapeDtypeStruct((NCORES, n_pad, H), jnp.float32),
            jax.ShapeDtypeStruct((NCORES * n_pad,), jnp.float32),
        ],
        scratch_types=[
            pltpu.VMEM((CHUNK, H // 2), jnp.int32),
            pltpu.VMEM((CHUNK, H // 2), jnp.int32),
            pltpu.VMEM((CHUNK, H), jnp.float32),
            pltpu.VMEM((CHUNK, H), jnp.float32),
            pltpu.VMEM((CHUNK, H), jnp.float32),
            pltpu.VMEM((CHUNK, H), jnp.float32),
            pltpu.VMEM((CHUNK, H), jnp.float32),
            pltpu.VMEM((CHUNK, H), jnp.float32),
            pltpu.VMEM((GC, CHUNK), jnp.int32),
            pltpu.VMEM((GC, CHUNK), jnp.int32),
            pltpu.VMEM(((CHUNK // L + 1) * L,), jnp.float32),
            pltpu.VMEM_SHARED((n_pad, H), jnp.float32),
            pltpu.VMEM_SHARED((n_pad,), jnp.float32),
            pltpu.SemaphoreType.DMA,
            pltpu.SemaphoreType.DMA,
            pltpu.SemaphoreType.DMA,
            pltpu.SemaphoreType.DMA,
            pltpu.SemaphoreType.DMA,
            pltpu.SemaphoreType.DMA,
            pltpu.SemaphoreType.DMA,
            pltpu.SemaphoreType.DMA,
            pltpu.SemaphoreType.DMA,
            pltpu.SemaphoreType.DMA,
            pltpu.SemaphoreType.DMA,
            pltpu.SemaphoreType.DMA,
        ],
    )
    def sc_k(p_hbm, q_hbm, row_hbm, col_hbm, acc_out, cnt_out,
             q0, q1, a0, a1, a2, b0, b1, b2, ir2d, ic2d, ones,
             acc_sh, cnt_sh, sem_i, sem_q0, sem_q1,
             sem_a0, sem_a1, sem_a2, sem_b0, sem_b1, sem_b2,
             sem_s0, sem_s1, sem_s2):
        cid = lax.axis_index("c")
        sid = lax.axis_index("s")
        wid = cid * NSUB + sid
        ebase = wid * per_w
        r0 = sid * rows_per_tile
        qsets = ((q0, sem_q0), (q1, sem_q1))
        absets = ((a0, b0, sem_a0, sem_b0, sem_s0),
                  (a1, b1, sem_a1, sem_b1, sem_s1),
                  (a2, b2, sem_a2, sem_b2, sem_s2))

        def _zfill(e, carry):
            for g in range(H // L):
                a0[e, pl.ds(g * L, L)] = jnp.zeros((L,), jnp.float32)
            return carry
        lax.fori_loop(0, CHUNK, _zfill, 0)

        def _ofill(e, carry):
            ones[pl.ds(e * L, L)] = jnp.ones((L,), jnp.float32)
            return carry
        lax.fori_loop(0, CHUNK // L + 1, _ofill, 0)

        n_z = rows_per_tile // CHUNK
        for z in range(n_z):
            pltpu.sync_copy(a0, acc_sh.at[pl.ds(r0 + z * CHUNK, CHUNK)])
        rz = rows_per_tile - n_z * CHUNK
        if rz:
            pltpu.sync_copy(a0.at[pl.ds(0, rz)],
                            acc_sh.at[pl.ds(r0 + n_z * CHUNK, rz)])
        for z in range(-(-rows_per_tile // H)):
            nr = min(H, rows_per_tile - z * H)
            pltpu.sync_copy(a0.at[0, pl.ds(0, nr)],
                            cnt_sh.at[pl.ds(r0 + z * H, nr)])
        plsc.subcore_barrier()

        def _start_q(gbase, j):
            qb, sq = qsets[j % 2]
            return (pltpu.async_copy(
                q_hbm.at[pl.ds(gbase + j * CHUNK, CHUNK)], qb, sq),)

        def _start_ab(gbase, j):
            ab, bb, sa, sb, _ = absets[j % 3]
            ca = pltpu.async_copy(p_hbm.at[ir2d.at[j]], ab, sa)
            cb = pltpu.async_copy(p_hbm.at[ic2d.at[j]], bb, sb)
            return (ca, cb)

        def _group(g, carry):
            gbase = ebase + g * (CHUNK * GC)
            idx_cps = []
            for j in range(GC):
                idx_cps.append(pltpu.async_copy(
                    row_hbm.at[pl.ds(gbase + j * CHUNK, CHUNK)],
                    ir2d.at[j], sem_i))
                idx_cps.append(pltpu.async_copy(
                    col_hbm.at[pl.ds(gbase + j * CHUNK, CHUNK)],
                    ic2d.at[j], sem_i))
            for cp in idx_cps:
                cp.wait()

            pend_q = [(), ()]
            pend_ab = [(), (), ()]
            pend_sc = [(), (), ()]
            pend_q[0] = _start_q(gbase, 0)
            pend_ab[0] = _start_ab(gbase, 0)
            pend_ab[1] = _start_ab(gbase, 1)
            for j in range(GC):
                s = j % 3
                qb, _ = qsets[j % 2]
                ab, bb, _, _, sem_s = absets[s]
                for cp in pend_q[j % 2]:
                    cp.wait()
                pend_q[j % 2] = ()
                for cp in pend_ab[s]:
                    cp.wait()
                pend_ab[s] = ()
                if j + 1 < GC:
                    pend_q[(j + 1) % 2] = _start_q(gbase, j + 1)
                if j + 2 < GC:
                    nxt = (j + 2) % 3
                    for cp in pend_sc[nxt]:
                        cp.wait()
                    pend_sc[nxt] = ()
                    pend_ab[nxt] = _start_ab(gbase, j + 2)

                def body(e, carry, qb=qb, ab=ab, bb=bb):
                    msk = jnp.int32(-65536)
                    for k in range(H // 2 // L):
                        wq = qb[e, pl.ds(k * L, L)]
                        qlo = lax.bitcast_convert_type(wq << 16, jnp.float32)
                        qhi = lax.bitcast_convert_type(wq & msk, jnp.float32)
                        slo = pl.ds(k * L, L)
                        shi = pl.ds(H // 2 + k * L, L)
                        ab[e, slo] = jnp.maximum(ab[e, slo] + qlo, 0.0)
                        ab[e, shi] = jnp.maximum(ab[e, shi] + qhi, 0.0)
                        bb[e, slo] = jnp.maximum(bb[e, slo] + qlo, 0.0)
                        bb[e, shi] = jnp.maximum(bb[e, shi] + qhi, 0.0)
                    return carry
                lax.fori_loop(0, CHUNK, body, 0)

                pend_sc[s] = (
                    pltpu.async_copy(ab, acc_sh.at[ic2d.at[j]], sem_s,
                                     add=True),
                    pltpu.async_copy(bb, acc_sh.at[ir2d.at[j]], sem_s,
                                     add=True),
                    pltpu.async_copy(ones.at[pl.ds(0, CHUNK)],
                                     cnt_sh.at[ic2d.at[j]], sem_s, add=True),
                    pltpu.async_copy(ones.at[pl.ds(0, CHUNK)],
                                     cnt_sh.at[ir2d.at[j]], sem_s, add=True),
                )
            for grp in pend_sc:
                for cp in grp:
                    cp.wait()
            return carry
        lax.fori_loop(0, n_groups, _group, 0)

        plsc.subcore_barrier()
        pltpu.sync_copy(acc_sh.at[pl.ds(r0, rows_per_tile)],
                        acc_out.at[cid, pl.ds(r0, rows_per_tile)])
        for z in range(-(-rows_per_tile // H)):
            nr = min(H, rows_per_tile - z * H)
            pltpu.sync_copy(cnt_sh.at[pl.ds(r0 + z * H, nr)],
                            a0.at[z, pl.ds(0, nr)])
            pltpu.sync_copy(a0.at[z, pl.ds(0, nr)],
                            cnt_out.at[pl.ds(cid * n_pad + r0 + z * H, nr)])

    return sc_k


def _fin_body(acc_ref, cnt_ref, nodes_ref, sc_ref, bi_ref, o_ref):
    acc = acc_ref[0] + acc_ref[1]
    cnt = cnt_ref[...]
    u = acc / jnp.maximum(cnt, 1.0) + nodes_ref[...]
    mu = jnp.mean(u, axis=1, keepdims=True)
    var = jnp.mean((u - mu) ** 2, axis=1, keepdims=True)
    o_ref[...] = (u - mu) * lax.rsqrt(var + 1e-6) * sc_ref[...] + bi_ref[...]


def _finalize(acc2, cnt1, nodes, scale2d, bias2d, block_rows):
    n = nodes.shape[0]
    assert n % block_rows == 0
    return pl.pallas_call(
        _fin_body,
        grid=(n // block_rows,),
        in_specs=[
            pl.BlockSpec((NCORES, block_rows, H), lambda i: (0, i, 0)),
            pl.BlockSpec((block_rows, 1), lambda i: (i, 0)),
            pl.BlockSpec((block_rows, H), lambda i: (i, 0)),
            pl.BlockSpec((1, H), lambda i: (0, 0)),
            pl.BlockSpec((1, H), lambda i: (0, 0)),
        ],
        out_specs=pl.BlockSpec((block_rows, H), lambda i: (i, 0)),
        out_shape=jax.ShapeDtypeStruct((n, H), jnp.float32),
    )(acc2, cnt1, nodes, scale2d, bias2d)


def kernel(node_features, edge_index, edge_features, W_msg, b_msg, ln_scale, ln_bias):
    nodes = node_features[0]
    feat = edge_features[0]
    row = edge_index[0, 0].astype(jnp.int32)
    col = edge_index[0, 1].astype(jnp.int32)
    n_nodes = nodes.shape[0]
    n_edges = feat.shape[0]

    b2d = b_msg.reshape(1, H)
    zb = jnp.zeros((1, H), jnp.float32)

    p = _mm_bias(nodes, W_msg, 0, b2d, 1000, packed=False)
    q = _mm_bias(feat, W_msg, 1, zb, 2000, packed=True)

    acc2, cnt2 = _sc_edge_kernel(n_nodes, n_edges)(p, q, row, col)
    n_pad = acc2.shape[1]
    cnt1 = (cnt2[:n_pad] + cnt2[n_pad:])[:, None]

    out = _finalize(acc2, cnt1, nodes, ln_scale.reshape(1, H),
                    ln_bias.reshape(1, H), 2000)
    return out[None]


# speedup vs baseline: 1.0813x; 1.0246x over previous
import functools

import jax
import jax.numpy as jnp
from jax import lax
from jax.experimental import pallas as pl
from jax.experimental.pallas import tpu as pltpu
from jax.experimental.pallas import tpu_sc as plsc

H = 128
L = 16
NCORES = 2
NSUB = 16
NW = NCORES * NSUB
CHUNK = 40
GC = 25


def _mm_body_f32(x_ref, w_ref, b_ref, o_ref):
    o_ref[...] = (
        jnp.dot(x_ref[...], w_ref[...], preferred_element_type=jnp.float32)
        + b_ref[...]
    )


def _mm_body_packed(x_ref, w_ref, b_ref, o_ref):
    y = (jnp.dot(x_ref[...], w_ref[...], preferred_element_type=jnp.float32)
         + b_ref[...])
    lo = lax.bitcast_convert_type(y[:, :H // 2].astype(jnp.bfloat16),
                                  jnp.uint16)
    hi = lax.bitcast_convert_type(y[:, H // 2:].astype(jnp.bfloat16),
                                  jnp.uint16)
    word = lo.astype(jnp.uint32) | (hi.astype(jnp.uint32) << 16)
    o_ref[...] = lax.bitcast_convert_type(word, jnp.int32)


def _mm_bias(x, w_full, w_half, b2d, block_rows, packed):
    m = x.shape[0]
    assert m % block_rows == 0
    body = _mm_body_packed if packed else _mm_body_f32
    ow = H // 2 if packed else H
    odt = jnp.int32 if packed else jnp.float32
    return pl.pallas_call(
        body,
        grid=(m // block_rows,),
        in_specs=[
            pl.BlockSpec((block_rows, H), lambda i: (i, 0)),
            pl.BlockSpec((H, H), lambda i, h=w_half: (h, 0)),
            pl.BlockSpec((1, H), lambda i: (0, 0)),
        ],
        out_specs=pl.BlockSpec((block_rows, ow), lambda i: (i, 0)),
        out_shape=jax.ShapeDtypeStruct((m, ow), odt),
    )(x, w_full, b2d)


def _sc_edge_kernel(n_nodes, n_edges):
    per_w = n_edges // NW
    n_groups = per_w // (CHUNK * GC)
    assert per_w == n_groups * CHUNK * GC, "edge partition must be exact"
    rows_per_tile = -(-n_nodes // NSUB)
    rows_per_tile = -(-rows_per_tile // 8) * 8
    n_pad = rows_per_tile * NSUB

    mesh = plsc.VectorSubcoreMesh(core_axis_name="c", subcore_axis_name="s")

    @functools.partial(
        pl.kernel,
        mesh=mesh,
        out_type=[
            jax.ShapeDtypeStruct((NCORES, n_pad, H), jnp.float32),
            jax.ShapeDtypeStruct((NCORES * n_pad,), jnp.float32),
        ],
        scratch_types=[
            pltpu.VMEM((CHUNK, H // 2), jnp.int32),
            pltpu.VMEM((CHUNK, H // 2), jnp.int32),
            pltpu.VMEM((2 * CHUNK, H), jnp.float32),
            pltpu.VMEM((2 * CHUNK, H), jnp.float32),
            pltpu.VMEM((2 * CHUNK, H), jnp.float32),
            pltpu.VMEM((GC, 2 * CHUNK), jnp.int32),
            pltpu.VMEM((2 * CHUNK,), jnp.float32),
            pltpu.VMEM_SHARED((n_pad, H), jnp.float32),
            pltpu.VMEM_SHARED((n_pad,), jnp.float32),
            pltpu.SemaphoreType.DMA,
            pltpu.SemaphoreType.DMA,
            pltpu.SemaphoreType.DMA,
            pltpu.SemaphoreType.DMA,
            pltpu.SemaphoreType.DMA,
            pltpu.SemaphoreType.DMA,
            pltpu.SemaphoreType.DMA,
            pltpu.SemaphoreType.DMA,
            pltpu.SemaphoreType.DMA,
        ],
    )
    def sc_k(p_hbm, q_hbm, row_hbm, col_hbm, acc_out, cnt_out,
             q0, q1, m0, m1, m2, icr2d, ones,
             acc_sh, cnt_sh, sem_i, sem_q0, sem_q1,
             sem_g0, sem_g1, sem_g2, sem_s0, sem_s1, sem_s2):
        cid = lax.axis_index("c")
        sid = lax.axis_index("s")
        wid = cid * NSUB + sid
        ebase = wid * per_w
        r0 = sid * rows_per_tile
        qsets = ((q0, sem_q0), (q1, sem_q1))
        msets = ((m0, sem_g0, sem_s0),
                 (m1, sem_g1, sem_s1),
                 (m2, sem_g2, sem_s2))

        def _zfill(e, carry):
            for g in range(H // L):
                m0[e, pl.ds(g * L, L)] = jnp.zeros((L,), jnp.float32)
            return carry
        lax.fori_loop(0, CHUNK, _zfill, 0)

        def _ofill(e, carry):
            ones[pl.ds(e * L, L)] = jnp.ones((L,), jnp.float32)
            return carry
        lax.fori_loop(0, 2 * CHUNK // L, _ofill, 0)

        n_z = rows_per_tile // CHUNK
        for z in range(n_z):
            pltpu.sync_copy(m0.at[pl.ds(0, CHUNK)],
                            acc_sh.at[pl.ds(r0 + z * CHUNK, CHUNK)])
        rz = rows_per_tile - n_z * CHUNK
        if rz:
            pltpu.sync_copy(m0.at[pl.ds(0, rz)],
                            acc_sh.at[pl.ds(r0 + n_z * CHUNK, rz)])
        for z in range(-(-rows_per_tile // H)):
            nr = min(H, rows_per_tile - z * H)
            pltpu.sync_copy(m0.at[0, pl.ds(0, nr)],
                            cnt_sh.at[pl.ds(r0 + z * H, nr)])
        plsc.subcore_barrier()

        def _start_q(gbase, j):
            qb, sq = qsets[j % 2]
            return (pltpu.async_copy(
                q_hbm.at[pl.ds(gbase + j * CHUNK, CHUNK)], qb, sq),)

        def _start_ab(gbase, j):
            mb, sg, _ = msets[j % 3]
            ca = pltpu.async_copy(p_hbm.at[icr2d.at[j, pl.ds(CHUNK, CHUNK)]],
                                  mb.at[pl.ds(0, CHUNK)], sg)
            cb = pltpu.async_copy(p_hbm.at[icr2d.at[j, pl.ds(0, CHUNK)]],
                                  mb.at[pl.ds(CHUNK, CHUNK)], sg)
            return (ca, cb)

        def _group(g, carry):
            gbase = ebase + g * (CHUNK * GC)
            idx_cps = []
            for j in range(GC):
                idx_cps.append(pltpu.async_copy(
                    col_hbm.at[pl.ds(gbase + j * CHUNK, CHUNK)],
                    icr2d.at[j, pl.ds(0, CHUNK)], sem_i))
                idx_cps.append(pltpu.async_copy(
                    row_hbm.at[pl.ds(gbase + j * CHUNK, CHUNK)],
                    icr2d.at[j, pl.ds(CHUNK, CHUNK)], sem_i))
            for cp in idx_cps:
                cp.wait()

            pend_q = [(), ()]
            pend_ab = [(), (), ()]
            pend_sc = [(), (), ()]
            pend_q[0] = _start_q(gbase, 0)
            pend_ab[0] = _start_ab(gbase, 0)
            pend_ab[1] = _start_ab(gbase, 1)
            for j in range(GC):
                s = j % 3
                qb, _ = qsets[j % 2]
                mb, _, sem_s = msets[s]
                for cp in pend_q[j % 2]:
                    cp.wait()
                pend_q[j % 2] = ()
                for cp in pend_ab[s]:
                    cp.wait()
                pend_ab[s] = ()
                if j + 1 < GC:
                    pend_q[(j + 1) % 2] = _start_q(gbase, j + 1)
                if j + 2 < GC:
                    nxt = (j + 2) % 3
                    for cp in pend_sc[nxt]:
                        cp.wait()
                    pend_sc[nxt] = ()
                    pend_ab[nxt] = _start_ab(gbase, j + 2)

                def body(e, carry, qb=qb, mb=mb):
                    msk = jnp.int32(-65536)
                    for k in range(H // 2 // L):
                        wq = qb[e, pl.ds(k * L, L)]
                        qlo = lax.bitcast_convert_type(wq << 16, jnp.float32)
                        qhi = lax.bitcast_convert_type(wq & msk, jnp.float32)
                        slo = pl.ds(k * L, L)
                        shi = pl.ds(H // 2 + k * L, L)
                        mb[e, slo] = jnp.maximum(mb[e, slo] + qlo, 0.0)
                        mb[e, shi] = jnp.maximum(mb[e, shi] + qhi, 0.0)
                        e2 = e + CHUNK
                        mb[e2, slo] = jnp.maximum(mb[e2, slo] + qlo, 0.0)
                        mb[e2, shi] = jnp.maximum(mb[e2, shi] + qhi, 0.0)
                    return carry
                lax.fori_loop(0, CHUNK, body, 0)

                pend_sc[s] = (
                    pltpu.async_copy(mb, acc_sh.at[icr2d.at[j]], sem_s,
                                     add=True),
                    pltpu.async_copy(ones, cnt_sh.at[icr2d.at[j]], sem_s,
                                     add=True),
                )
            for grp in pend_sc:
                for cp in grp:
                    cp.wait()
            return carry
        lax.fori_loop(0, n_groups, _group, 0)

        plsc.subcore_barrier()
        pltpu.sync_copy(acc_sh.at[pl.ds(r0, rows_per_tile)],
                        acc_out.at[cid, pl.ds(r0, rows_per_tile)])
        for z in range(-(-rows_per_tile // H)):
            nr = min(H, rows_per_tile - z * H)
            pltpu.sync_copy(cnt_sh.at[pl.ds(r0 + z * H, nr)],
                            m0.at[z, pl.ds(0, nr)])
            pltpu.sync_copy(m0.at[z, pl.ds(0, nr)],
                            cnt_out.at[pl.ds(cid * n_pad + r0 + z * H, nr)])

    return sc_k


def _fin_body(acc_ref, cnt_ref, nodes_ref, sc_ref, bi_ref, o_ref):
    acc = acc_ref[0] + acc_ref[1]
    cnt = cnt_ref[...]
    u = acc / jnp.maximum(cnt, 1.0) + nodes_ref[...]
    mu = jnp.mean(u, axis=1, keepdims=True)
    var = jnp.mean((u - mu) ** 2, axis=1, keepdims=True)
    o_ref[...] = (u - mu) * lax.rsqrt(var + 1e-6) * sc_ref[...] + bi_ref[...]


def _finalize(acc2, cnt1, nodes, scale2d, bias2d, block_rows):
    n = nodes.shape[0]
    assert n % block_rows == 0
    return pl.pallas_call(
        _fin_body,
        grid=(n // block_rows,),
        in_specs=[
            pl.BlockSpec((NCORES, block_rows, H), lambda i: (0, i, 0)),
            pl.BlockSpec((block_rows, 1), lambda i: (i, 0)),
            pl.BlockSpec((block_rows, H), lambda i: (i, 0)),
            pl.BlockSpec((1, H), lambda i: (0, 0)),
            pl.BlockSpec((1, H), lambda i: (0, 0)),
        ],
        out_specs=pl.BlockSpec((block_rows, H), lambda i: (i, 0)),
        out_shape=jax.ShapeDtypeStruct((n, H), jnp.float32),
    )(acc2, cnt1, nodes, scale2d, bias2d)


def kernel(node_features, edge_index, edge_features, W_msg, b_msg, ln_scale, ln_bias):
    nodes = node_features[0]
    feat = edge_features[0]
    row = edge_index[0, 0].astype(jnp.int32)
    col = edge_index[0, 1].astype(jnp.int32)
    n_nodes = nodes.shape[0]
    n_edges = feat.shape[0]

    b2d = b_msg.reshape(1, H)
    zb = jnp.zeros((1, H), jnp.float32)

    p = _mm_bias(nodes, W_msg, 0, b2d, 1000, packed=False)
    q = _mm_bias(feat, W_msg, 1, zb, 2000, packed=True)

    acc2, cnt2 = _sc_edge_kernel(n_nodes, n_edges)(p, q, row, col)
    n_pad = acc2.shape[1]
    cnt1 = (cnt2[:n_pad] + cnt2[n_pad:])[:, None]

    out = _finalize(acc2, cnt1, nodes, ln_scale.reshape(1, H),
                    ln_bias.reshape(1, H), 2000)
    return out[None]


# speedup vs baseline: 1.1831x; 1.0941x over previous
import functools

import jax
import jax.numpy as jnp
from jax import lax
from jax.experimental import pallas as pl
from jax.experimental.pallas import tpu as pltpu
from jax.experimental.pallas import tpu_sc as plsc

H = 128
L = 16
NCORES = 2
NSUB = 16
NW = NCORES * NSUB
CHUNK = 40
GC = 25


def _mm_body_f32(x_ref, w_ref, b_ref, o_ref):
    o_ref[...] = (
        jnp.dot(x_ref[...], w_ref[...], preferred_element_type=jnp.float32)
        + b_ref[...]
    )


def _mm_body_packed(x_ref, w_ref, b_ref, o_ref):
    y = (jnp.dot(x_ref[...], w_ref[...], preferred_element_type=jnp.float32)
         + b_ref[...])
    lo = lax.bitcast_convert_type(y[:, :H // 2].astype(jnp.bfloat16),
                                  jnp.uint16)
    hi = lax.bitcast_convert_type(y[:, H // 2:].astype(jnp.bfloat16),
                                  jnp.uint16)
    word = lo.astype(jnp.uint32) | (hi.astype(jnp.uint32) << 16)
    o_ref[...] = lax.bitcast_convert_type(word, jnp.int32)


def _mm_bias(x, w_full, w_half, b2d, block_rows, packed, rows=None,
             row_off=0):
    m = x.shape[0] if rows is None else rows
    assert m % block_rows == 0 and row_off % block_rows == 0
    body = _mm_body_packed if packed else _mm_body_f32
    ow = H // 2 if packed else H
    odt = jnp.int32 if packed else jnp.float32
    ob = row_off // block_rows
    return pl.pallas_call(
        body,
        grid=(m // block_rows,),
        in_specs=[
            pl.BlockSpec((block_rows, H), lambda i, o=ob: (i + o, 0)),
            pl.BlockSpec((H, H), lambda i, h=w_half: (h, 0)),
            pl.BlockSpec((1, H), lambda i: (0, 0)),
        ],
        out_specs=pl.BlockSpec((block_rows, ow), lambda i: (i, 0)),
        out_shape=jax.ShapeDtypeStruct((m, ow), odt),
    )(x, w_full, b2d)


def _sc_edge_kernel(n_nodes, n_edges, edge_off):
    per_w = n_edges // NW
    n_groups = per_w // (CHUNK * GC)
    assert per_w == n_groups * CHUNK * GC, "edge partition must be exact"
    rows_per_tile = -(-n_nodes // NSUB)
    rows_per_tile = -(-rows_per_tile // 8) * 8
    n_pad = rows_per_tile * NSUB

    mesh = plsc.VectorSubcoreMesh(core_axis_name="c", subcore_axis_name="s")

    @functools.partial(
        pl.kernel,
        mesh=mesh,
        out_type=[
            jax.ShapeDtypeStruct((NCORES, n_pad, H), jnp.float32),
            jax.ShapeDtypeStruct((NCORES * n_pad,), jnp.float32),
        ],
        scratch_types=[
            pltpu.VMEM((CHUNK, H // 2), jnp.int32),
            pltpu.VMEM((CHUNK, H // 2), jnp.int32),
            pltpu.VMEM((2 * CHUNK, H), jnp.float32),
            pltpu.VMEM((2 * CHUNK, H), jnp.float32),
            pltpu.VMEM((2 * CHUNK, H), jnp.float32),
            pltpu.VMEM((GC, 2 * CHUNK), jnp.int32),
            pltpu.VMEM((2 * CHUNK,), jnp.float32),
            pltpu.VMEM_SHARED((n_pad, H), jnp.float32),
            pltpu.VMEM_SHARED((n_pad,), jnp.float32),
            pltpu.SemaphoreType.DMA,
            pltpu.SemaphoreType.DMA,
            pltpu.SemaphoreType.DMA,
            pltpu.SemaphoreType.DMA,
            pltpu.SemaphoreType.DMA,
            pltpu.SemaphoreType.DMA,
            pltpu.SemaphoreType.DMA,
            pltpu.SemaphoreType.DMA,
            pltpu.SemaphoreType.DMA,
        ],
    )
    def sc_k(p_hbm, q_hbm, row_hbm, col_hbm, acc_out, cnt_out,
             q0, q1, m0, m1, m2, icr2d, ones,
             acc_sh, cnt_sh, sem_i, sem_q0, sem_q1,
             sem_g0, sem_g1, sem_g2, sem_s0, sem_s1, sem_s2):
        cid = lax.axis_index("c")
        sid = lax.axis_index("s")
        wid = cid * NSUB + sid
        ebase = wid * per_w
        r0 = sid * rows_per_tile
        qsets = ((q0, sem_q0), (q1, sem_q1))
        msets = ((m0, sem_g0, sem_s0),
                 (m1, sem_g1, sem_s1),
                 (m2, sem_g2, sem_s2))

        def _zfill(e, carry):
            for g in range(H // L):
                m0[e, pl.ds(g * L, L)] = jnp.zeros((L,), jnp.float32)
            return carry
        lax.fori_loop(0, CHUNK, _zfill, 0)

        def _ofill(e, carry):
            ones[pl.ds(e * L, L)] = jnp.ones((L,), jnp.float32)
            return carry
        lax.fori_loop(0, 2 * CHUNK // L, _ofill, 0)

        n_z = rows_per_tile // CHUNK
        for z in range(n_z):
            pltpu.sync_copy(m0.at[pl.ds(0, CHUNK)],
                            acc_sh.at[pl.ds(r0 + z * CHUNK, CHUNK)])
        rz = rows_per_tile - n_z * CHUNK
        if rz:
            pltpu.sync_copy(m0.at[pl.ds(0, rz)],
                            acc_sh.at[pl.ds(r0 + n_z * CHUNK, rz)])
        for z in range(-(-rows_per_tile // H)):
            nr = min(H, rows_per_tile - z * H)
            pltpu.sync_copy(m0.at[0, pl.ds(0, nr)],
                            cnt_sh.at[pl.ds(r0 + z * H, nr)])
        plsc.subcore_barrier()

        def _start_q(gbase, j):
            qb, sq = qsets[j % 2]
            return (pltpu.async_copy(
                q_hbm.at[pl.ds(gbase + j * CHUNK, CHUNK)], qb, sq),)

        def _start_ab(gbase, j):
            mb, sg, _ = msets[j % 3]
            ca = pltpu.async_copy(p_hbm.at[icr2d.at[j, pl.ds(CHUNK, CHUNK)]],
                                  mb.at[pl.ds(0, CHUNK)], sg)
            cb = pltpu.async_copy(p_hbm.at[icr2d.at[j, pl.ds(0, CHUNK)]],
                                  mb.at[pl.ds(CHUNK, CHUNK)], sg)
            return (ca, cb)

        def _group(g, carry):
            gbase = ebase + g * (CHUNK * GC)
            idx_cps = []
            for j in range(GC):
                idx_cps.append(pltpu.async_copy(
                    col_hbm.at[pl.ds(edge_off + gbase + j * CHUNK, CHUNK)],
                    icr2d.at[j, pl.ds(0, CHUNK)], sem_i))
                idx_cps.append(pltpu.async_copy(
                    row_hbm.at[pl.ds(edge_off + gbase + j * CHUNK, CHUNK)],
                    icr2d.at[j, pl.ds(CHUNK, CHUNK)], sem_i))
            for cp in idx_cps:
                cp.wait()

            pend_q = [(), ()]
            pend_ab = [(), (), ()]
            pend_sc = [(), (), ()]
            pend_q[0] = _start_q(gbase, 0)
            pend_ab[0] = _start_ab(gbase, 0)
            pend_ab[1] = _start_ab(gbase, 1)
            for j in range(GC):
                s = j % 3
                qb, _ = qsets[j % 2]
                mb, _, sem_s = msets[s]
                for cp in pend_q[j % 2]:
                    cp.wait()
                pend_q[j % 2] = ()
                for cp in pend_ab[s]:
                    cp.wait()
                pend_ab[s] = ()
                if j + 1 < GC:
                    pend_q[(j + 1) % 2] = _start_q(gbase, j + 1)
                if j + 2 < GC:
                    nxt = (j + 2) % 3
                    for cp in pend_sc[nxt]:
                        cp.wait()
                    pend_sc[nxt] = ()
                    pend_ab[nxt] = _start_ab(gbase, j + 2)

                def body(e, carry, qb=qb, mb=mb):
                    msk = jnp.int32(-65536)
                    for k in range(H // 2 // L):
                        wq = qb[e, pl.ds(k * L, L)]
                        qlo = lax.bitcast_convert_type(wq << 16, jnp.float32)
                        qhi = lax.bitcast_convert_type(wq & msk, jnp.float32)
                        slo = pl.ds(k * L, L)
                        shi = pl.ds(H // 2 + k * L, L)
                        mb[e, slo] = jnp.maximum(mb[e, slo] + qlo, 0.0)
                        mb[e, shi] = jnp.maximum(mb[e, shi] + qhi, 0.0)
                        e2 = e + CHUNK
                        mb[e2, slo] = jnp.maximum(mb[e2, slo] + qlo, 0.0)
                        mb[e2, shi] = jnp.maximum(mb[e2, shi] + qhi, 0.0)
                    return carry
                lax.fori_loop(0, CHUNK, body, 0)

                pend_sc[s] = (
                    pltpu.async_copy(mb, acc_sh.at[icr2d.at[j]], sem_s,
                                     add=True),
                    pltpu.async_copy(ones, cnt_sh.at[icr2d.at[j]], sem_s,
                                     add=True),
                )
            for grp in pend_sc:
                for cp in grp:
                    cp.wait()
            return carry
        lax.fori_loop(0, n_groups, _group, 0)

        plsc.subcore_barrier()
        pltpu.sync_copy(acc_sh.at[pl.ds(r0, rows_per_tile)],
                        acc_out.at[cid, pl.ds(r0, rows_per_tile)])
        for z in range(-(-rows_per_tile // H)):
            nr = min(H, rows_per_tile - z * H)
            pltpu.sync_copy(cnt_sh.at[pl.ds(r0 + z * H, nr)],
                            m0.at[z, pl.ds(0, nr)])
            pltpu.sync_copy(m0.at[z, pl.ds(0, nr)],
                            cnt_out.at[pl.ds(cid * n_pad + r0 + z * H, nr)])

    return sc_k


def _fin_body(acc_ref, accb_ref, cnt_ref, nodes_ref, sc_ref, bi_ref, o_ref):
    acc = (acc_ref[0] + acc_ref[1]) + (accb_ref[0] + accb_ref[1])
    cnt = cnt_ref[...]
    u = acc / jnp.maximum(cnt, 1.0) + nodes_ref[...]
    mu = jnp.mean(u, axis=1, keepdims=True)
    var = jnp.mean((u - mu) ** 2, axis=1, keepdims=True)
    o_ref[...] = (u - mu) * lax.rsqrt(var + 1e-6) * sc_ref[...] + bi_ref[...]


def _finalize(acc2a, acc2b, cnt1, nodes, scale2d, bias2d, block_rows):
    n = nodes.shape[0]
    assert n % block_rows == 0
    return pl.pallas_call(
        _fin_body,
        grid=(n // block_rows,),
        in_specs=[
            pl.BlockSpec((NCORES, block_rows, H), lambda i: (0, i, 0)),
            pl.BlockSpec((NCORES, block_rows, H), lambda i: (0, i, 0)),
            pl.BlockSpec((block_rows, 1), lambda i: (i, 0)),
            pl.BlockSpec((block_rows, H), lambda i: (i, 0)),
            pl.BlockSpec((1, H), lambda i: (0, 0)),
            pl.BlockSpec((1, H), lambda i: (0, 0)),
        ],
        out_specs=pl.BlockSpec((block_rows, H), lambda i: (i, 0)),
        out_shape=jax.ShapeDtypeStruct((n, H), jnp.float32),
    )(acc2a, acc2b, cnt1, nodes, scale2d, bias2d)


def kernel(node_features, edge_index, edge_features, W_msg, b_msg, ln_scale, ln_bias):
    nodes = node_features[0]
    feat = edge_features[0]
    row = edge_index[0, 0].astype(jnp.int32)
    col = edge_index[0, 1].astype(jnp.int32)
    n_nodes = nodes.shape[0]
    n_edges = feat.shape[0]

    b2d = b_msg.reshape(1, H)
    zb = jnp.zeros((1, H), jnp.float32)

    p = _mm_bias(nodes, W_msg, 0, b2d, 1000, packed=False)
    eh = n_edges // 2
    q_a = _mm_bias(feat, W_msg, 1, zb, 2000, packed=True, rows=eh, row_off=0)
    q_b = _mm_bias(feat, W_msg, 1, zb, 2000, packed=True, rows=eh, row_off=eh)

    acc2a, cnt2a = _sc_edge_kernel(n_nodes, eh, 0)(p, q_a, row, col)
    acc2b, cnt2b = _sc_edge_kernel(n_nodes, eh, eh)(p, q_b, row, col)
    n_pad = acc2a.shape[1]
    cnt_all = cnt2a + cnt2b
    cnt1 = (cnt_all[:n_pad] + cnt_all[n_pad:])[:, None]

    out = _finalize(acc2a, acc2b, cnt1, nodes, ln_scale.reshape(1, H),
                    ln_bias.reshape(1, H), 2000)
    return out[None]
